# baseline (speedup 1.0000x reference)
import functools

import jax
import jax.numpy as jnp
from jax import lax
from jax.experimental import pallas as pl
from jax.experimental.pallas import tpu as pltpu
from jax.experimental.pallas import tpu_sc as plsc


def _proj_kernel(a_ref, embt_ref, out_ref):
    out_ref[...] = jnp.dot(a_ref[...], embt_ref[...],
                           preferred_element_type=jnp.float32
                           ).astype(jnp.bfloat16)


def _make_sc_gather(Vp, B, F):
    mesh = plsc.VectorSubcoreMesh(core_axis_name="c", subcore_axis_name="s")

    @functools.partial(
        pl.kernel,
        out_type=jax.ShapeDtypeStruct((F, B), jnp.float32),
        mesh=mesh,
        compiler_params=pltpu.CompilerParams(needs_layout_passes=False),
        scratch_types=[
            pltpu.VMEM((Vp // 2,), jnp.int32),
            pltpu.VMEM((B,), jnp.int32),
            pltpu.VMEM((B,), jnp.float32),
        ],
    )
    def sc_gather(projT_hbm, idxT_hbm, out_hbm, tab_v, idx_v, out_v):
        f = lax.axis_index("s") * 2 + lax.axis_index("c")

        @pl.when(f < F)
        def _():
            pltpu.sync_copy(projT_hbm.at[f], tab_v)
            pltpu.sync_copy(idxT_hbm.at[f], idx_v)

            def body(i, carry):
                ids = idx_v[pl.ds(i * 16, 16)]
                w = plsc.load_gather(tab_v, [lax.shift_right_logical(ids, 1)])
                sel = jnp.where((ids & 1) == 1,
                                lax.shift_right_logical(w, 16), w)
                out_v[pl.ds(i * 16, 16)] = plsc.bitcast(
                    lax.shift_left(sel, 16), jnp.float32)
                return carry

            lax.fori_loop(0, B // 16, body, 0)
            pltpu.sync_copy(out_v, out_hbm.at[f])

    return sc_gather


def _dense_kernel(xt_ref, w1_ref, b1_ref, w2_ref, b2_ref,
                  w3_ref, b3_ref, woh_ref, bo_ref, out_ref):
    h = jnp.maximum(
        lax.dot_general(xt_ref[...], w1_ref[...],
                        dimension_numbers=(((0,), (0,)), ((), ())),
                        preferred_element_type=jnp.float32) + b1_ref[...], 0.0)
    h = jnp.maximum(
        jnp.dot(h, w2_ref[...],
                preferred_element_type=jnp.float32) + b2_ref[...], 0.0)
    h = jnp.maximum(
        jnp.dot(h, w3_ref[...],
                preferred_element_type=jnp.float32) + b3_ref[...], 0.0)
    out_ref[...] = (lax.dot_general(
        woh_ref[...], h, dimension_numbers=(((0,), (1,)), ((), ())),
        preferred_element_type=jnp.float32) + bo_ref[0, 0])


def _combine_kernel(part_ref, dense_ref, out_ref):
    out_ref[...] = jnp.sum(part_ref[...], axis=0)[None, :] + dense_ref[...]


def kernel(dense_features, sparse_indices, emb_table, W1, b1, W2, b2, W3, b3,
           Wo, bo):
    B, DIN = dense_features.shape
    _, F = sparse_indices.shape
    V, D = emb_table.shape

    VC = 8192
    G = -(-V // VC)
    Vp = G * VC

    embT = emb_table.T
    idxT = sparse_indices.T
    xT = dense_features.T
    A = Wo[:F * D, 0].reshape(F, D)
    woh = Wo[F * D:, :]

    projT = pl.pallas_call(
        _proj_kernel,
        grid=(G,),
        in_specs=[pl.BlockSpec((F, D), lambda i: (0, 0)),
                  pl.BlockSpec((D, VC), lambda i: (0, i))],
        out_specs=pl.BlockSpec((F, VC), lambda i: (0, i)),
        out_shape=jax.ShapeDtypeStruct((F, Vp), jnp.bfloat16),
    )(A, embT)
    projT_i32 = lax.bitcast_convert_type(
        projT.reshape(F, Vp // 2, 2), jnp.int32)

    partials = _make_sc_gather(Vp, B, F)(projT_i32, idxT)

    blk = 1024
    densepart = pl.pallas_call(
        _dense_kernel,
        grid=(B // blk,),
        in_specs=[pl.BlockSpec((DIN, blk), lambda i: (0, i)),
                  pl.BlockSpec((DIN, 512), lambda i: (0, 0)),
                  pl.BlockSpec((1, 512), lambda i: (0, 0)),
                  pl.BlockSpec((512, 256), lambda i: (0, 0)),
                  pl.BlockSpec((1, 256), lambda i: (0, 0)),
                  pl.BlockSpec((256, D), lambda i: (0, 0)),
                  pl.BlockSpec((1, D), lambda i: (0, 0)),
                  pl.BlockSpec((D, 1), lambda i: (0, 0)),
                  pl.BlockSpec((1, 1), lambda i: (0, 0))],
        out_specs=pl.BlockSpec((1, blk), lambda i: (0, i)),
        out_shape=jax.ShapeDtypeStruct((1, B), jnp.float32),
    )(xT, W1, b1.reshape(1, -1), W2, b2.reshape(1, -1),
      W3, b3.reshape(1, -1), woh, bo.reshape(1, 1))

    out = pl.pallas_call(
        _combine_kernel,
        grid=(B // blk,),
        in_specs=[pl.BlockSpec((F, blk), lambda i: (0, i)),
                  pl.BlockSpec((1, blk), lambda i: (0, i))],
        out_specs=pl.BlockSpec((1, blk), lambda i: (0, i)),
        out_shape=jax.ShapeDtypeStruct((1, B), jnp.float32),
    )(partials, densepart)
    return out.T


# speedup vs baseline: 9.2284x; 9.2284x over previous
import functools

import jax
import jax.numpy as jnp
from jax import lax
from jax.experimental import pallas as pl
from jax.experimental.pallas import tpu as pltpu
from jax.experimental.pallas import tpu_sc as plsc


def _proj_kernel(a_ref, emblo_ref, embhi_ref, out_ref):
    lo = jnp.dot(a_ref[...], emblo_ref[...],
                 preferred_element_type=jnp.float32).astype(jnp.bfloat16)
    hi = jnp.dot(a_ref[...], embhi_ref[...],
                 preferred_element_type=jnp.float32).astype(jnp.bfloat16)
    lo32 = lax.convert_element_type(
        lax.bitcast_convert_type(lo, jnp.uint16), jnp.int32)
    hi32 = lax.convert_element_type(
        lax.bitcast_convert_type(hi, jnp.uint16), jnp.int32)
    out_ref[...] = lax.shift_left(hi32, 16) | lo32


def _make_sc_gather(H, B, F):
    mesh = plsc.VectorSubcoreMesh(core_axis_name="c", subcore_axis_name="s")

    @functools.partial(
        pl.kernel,
        out_type=jax.ShapeDtypeStruct((F, B), jnp.float32),
        mesh=mesh,
        compiler_params=pltpu.CompilerParams(needs_layout_passes=False),
        scratch_types=[
            pltpu.VMEM((H,), jnp.int32),
            pltpu.VMEM((B,), jnp.int32),
            pltpu.VMEM((B,), jnp.float32),
        ],
    )
    def sc_gather(projT_hbm, idxT_hbm, out_hbm, tab_v, idx_v, out_v):
        f = lax.axis_index("s") * 2 + lax.axis_index("c")

        @pl.when(f < F)
        def _():
            pltpu.sync_copy(projT_hbm.at[f], tab_v)
            pltpu.sync_copy(idxT_hbm.at[f], idx_v)

            def body(i, carry):
                ids = idx_v[pl.ds(i * 16, 16)]
                in_lo = ids < H
                widx = jnp.where(in_lo, ids, ids - H)
                w = plsc.load_gather(tab_v, [widx])
                sel = jnp.where(in_lo, w, lax.shift_right_logical(w, 16))
                out_v[pl.ds(i * 16, 16)] = plsc.bitcast(
                    lax.shift_left(sel, 16), jnp.float32)
                return carry

            lax.fori_loop(0, B // 16, body, 0)
            pltpu.sync_copy(out_v, out_hbm.at[f])

    return sc_gather


def _dense_kernel(xt_ref, w1_ref, b1_ref, w2_ref, b2_ref,
                  w3_ref, b3_ref, woh_ref, bo_ref, out_ref):
    h = jnp.maximum(
        lax.dot_general(xt_ref[...], w1_ref[...],
                        dimension_numbers=(((0,), (0,)), ((), ())),
                        preferred_element_type=jnp.float32) + b1_ref[...], 0.0)
    h = jnp.maximum(
        jnp.dot(h, w2_ref[...],
                preferred_element_type=jnp.float32) + b2_ref[...], 0.0)
    h = jnp.maximum(
        jnp.dot(h, w3_ref[...],
                preferred_element_type=jnp.float32) + b3_ref[...], 0.0)
    out_ref[...] = (lax.dot_general(
        woh_ref[...], h, dimension_numbers=(((0,), (1,)), ((), ())),
        preferred_element_type=jnp.float32) + bo_ref[0, 0])


def _combine_kernel(part_ref, dense_ref, out_ref):
    out_ref[...] = jnp.sum(part_ref[...], axis=0)[None, :] + dense_ref[...]


def kernel(dense_features, sparse_indices, emb_table, W1, b1, W2, b2, W3, b3,
           Wo, bo):
    B, DIN = dense_features.shape
    _, F = sparse_indices.shape
    V, D = emb_table.shape

    VC = 5120
    G2 = -(-V // (2 * VC))
    H = G2 * VC

    embT = emb_table.T
    idxT = sparse_indices.T
    xT = dense_features.T
    A = Wo[:F * D, 0].reshape(F, D)
    woh = Wo[F * D:, :]

    projT = pl.pallas_call(
        _proj_kernel,
        grid=(G2,),
        in_specs=[pl.BlockSpec((F, D), lambda i: (0, 0)),
                  pl.BlockSpec((D, VC), lambda i: (0, i)),
                  pl.BlockSpec((D, VC), lambda i, _G2=G2: (0, i + _G2))],
        out_specs=pl.BlockSpec((F, VC), lambda i: (0, i)),
        out_shape=jax.ShapeDtypeStruct((F, H), jnp.int32),
    )(A, embT, embT)

    partials = _make_sc_gather(H, B, F)(projT, idxT)

    blk = 1024
    densepart = pl.pallas_call(
        _dense_kernel,
        grid=(B // blk,),
        in_specs=[pl.BlockSpec((DIN, blk), lambda i: (0, i)),
                  pl.BlockSpec((DIN, 512), lambda i: (0, 0)),
                  pl.BlockSpec((1, 512), lambda i: (0, 0)),
                  pl.BlockSpec((512, 256), lambda i: (0, 0)),
                  pl.BlockSpec((1, 256), lambda i: (0, 0)),
                  pl.BlockSpec((256, D), lambda i: (0, 0)),
                  pl.BlockSpec((1, D), lambda i: (0, 0)),
                  pl.BlockSpec((D, 1), lambda i: (0, 0)),
                  pl.BlockSpec((1, 1), lambda i: (0, 0))],
        out_specs=pl.BlockSpec((1, blk), lambda i: (0, i)),
        out_shape=jax.ShapeDtypeStruct((1, B), jnp.float32),
    )(xT, W1, b1.reshape(1, -1), W2, b2.reshape(1, -1),
      W3, b3.reshape(1, -1), woh, bo.reshape(1, 1))

    out = pl.pallas_call(
        _combine_kernel,
        grid=(B // blk,),
        in_specs=[pl.BlockSpec((F, blk), lambda i: (0, i)),
                  pl.BlockSpec((1, blk), lambda i: (0, i))],
        out_specs=pl.BlockSpec((1, blk), lambda i: (0, i)),
        out_shape=jax.ShapeDtypeStruct((1, B), jnp.float32),
    )(partials, densepart)
    return out.T


# speedup vs baseline: 10.0751x; 1.0917x over previous
import functools

import jax
import jax.numpy as jnp
from jax import lax
from jax.experimental import pallas as pl
from jax.experimental.pallas import tpu as pltpu
from jax.experimental.pallas import tpu_sc as plsc


def _proj_kernel(a_ref, emblo_ref, embhi_ref, out_ref):
    lo = jnp.dot(a_ref[...], emblo_ref[...],
                 preferred_element_type=jnp.float32).astype(jnp.bfloat16)
    hi = jnp.dot(a_ref[...], embhi_ref[...],
                 preferred_element_type=jnp.float32).astype(jnp.bfloat16)
    lo32 = lax.convert_element_type(
        lax.bitcast_convert_type(lo, jnp.uint16), jnp.int32)
    hi32 = lax.convert_element_type(
        lax.bitcast_convert_type(hi, jnp.uint16), jnp.int32)
    out_ref[...] = lax.shift_left(hi32, 16) | lo32


def _make_sc_gather(H, B, F):
    mesh = plsc.VectorSubcoreMesh(core_axis_name="c", subcore_axis_name="s")

    @functools.partial(
        pl.kernel,
        out_type=jax.ShapeDtypeStruct((F, B), jnp.float32),
        mesh=mesh,
        compiler_params=pltpu.CompilerParams(needs_layout_passes=False),
        scratch_types=[
            pltpu.VMEM((H,), jnp.int32),
            pltpu.VMEM((B,), jnp.int32),
            pltpu.VMEM((B,), jnp.float32),
        ],
    )
    def sc_gather(projT_hbm, idxT_hbm, out_hbm, tab_v, idx_v, out_v):
        f = lax.axis_index("s") * 2 + lax.axis_index("c")

        @pl.when(f < F)
        def _():
            pltpu.sync_copy(projT_hbm.at[f], tab_v)
            pltpu.sync_copy(idxT_hbm.at[f], idx_v)

            def body(i, carry):
                ids = idx_v[pl.ds(i * 16, 16)]
                in_lo = ids < H
                widx = jnp.where(in_lo, ids, ids - H)
                w = plsc.load_gather(tab_v, [widx])
                sel = jnp.where(in_lo, w, lax.shift_right_logical(w, 16))
                out_v[pl.ds(i * 16, 16)] = plsc.bitcast(
                    lax.shift_left(sel, 16), jnp.float32)
                return carry

            lax.fori_loop(0, B // 16, body, 0)
            pltpu.sync_copy(out_v, out_hbm.at[f])

    return sc_gather


def _dense_kernel(xt_ref, w1_ref, b1_ref, w2_ref, b2_ref,
                  w3_ref, b3_ref, woh_ref, bo_ref, out_ref):
    h = jnp.maximum(
        lax.dot_general(xt_ref[...], w1_ref[...],
                        dimension_numbers=(((0,), (0,)), ((), ())),
                        preferred_element_type=jnp.float32) + b1_ref[...], 0.0)
    h = jnp.maximum(
        jnp.dot(h, w2_ref[...],
                preferred_element_type=jnp.float32) + b2_ref[...], 0.0)
    h = jnp.maximum(
        jnp.dot(h, w3_ref[...],
                preferred_element_type=jnp.float32) + b3_ref[...], 0.0)
    out_ref[...] = (lax.dot_general(
        woh_ref[...], h, dimension_numbers=(((0,), (1,)), ((), ())),
        preferred_element_type=jnp.float32) + bo_ref[0, 0])


def _combine_kernel(part_ref, dense_ref, out_ref):
    out_ref[...] = jnp.sum(part_ref[...], axis=0)[None, :] + dense_ref[...]


def kernel(dense_features, sparse_indices, emb_table, W1, b1, W2, b2, W3, b3,
           Wo, bo):
    B, DIN = dense_features.shape
    _, F = sparse_indices.shape
    V, D = emb_table.shape

    VC = 10240
    G2 = -(-V // (2 * VC))
    H = G2 * VC

    embT = emb_table.T
    idxT = sparse_indices.T
    xT = dense_features.T
    A = Wo[:F * D, 0].reshape(F, D)
    woh = Wo[F * D:, :]

    projT = pl.pallas_call(
        _proj_kernel,
        grid=(G2,),
        in_specs=[pl.BlockSpec((F, D), lambda i: (0, 0)),
                  pl.BlockSpec((D, VC), lambda i: (0, i)),
                  pl.BlockSpec((D, VC), lambda i, _G2=G2: (0, i + _G2))],
        out_specs=pl.BlockSpec((F, VC), lambda i: (0, i)),
        out_shape=jax.ShapeDtypeStruct((F, H), jnp.int32),
    )(A, embT, embT)

    partials = _make_sc_gather(H, B, F)(projT, idxT)

    blk = 1024
    densepart = pl.pallas_call(
        _dense_kernel,
        grid=(B // blk,),
        in_specs=[pl.BlockSpec((DIN, blk), lambda i: (0, i)),
                  pl.BlockSpec((DIN, 512), lambda i: (0, 0)),
                  pl.BlockSpec((1, 512), lambda i: (0, 0)),
                  pl.BlockSpec((512, 256), lambda i: (0, 0)),
                  pl.BlockSpec((1, 256), lambda i: (0, 0)),
                  pl.BlockSpec((256, D), lambda i: (0, 0)),
                  pl.BlockSpec((1, D), lambda i: (0, 0)),
                  pl.BlockSpec((D, 1), lambda i: (0, 0)),
                  pl.BlockSpec((1, 1), lambda i: (0, 0))],
        out_specs=pl.BlockSpec((1, blk), lambda i: (0, i)),
        out_shape=jax.ShapeDtypeStruct((1, B), jnp.float32),
    )(xT, W1, b1.reshape(1, -1), W2, b2.reshape(1, -1),
      W3, b3.reshape(1, -1), woh, bo.reshape(1, 1))

    out = pl.pallas_call(
        _combine_kernel,
        out_shape=jax.ShapeDtypeStruct((1, B), jnp.float32),
    )(partials, densepart)
    return out.T
